# baseline (speedup 1.0000x reference)
import functools

import jax
import jax.numpy as jnp
from jax import lax
from jax.experimental import pallas as pl
from jax.experimental.pallas import tpu as pltpu
from jax.experimental.pallas import tpu_sc as plsc

N = 10000
E = 320000
D = 128
D_OUT = 64

NC = 2
NS = 16
NW = NC * NS
B = 80
CPT = -(-E // (NW * B))
C_EDGES = CPT * B
E_PAD = NW * C_EDGES
NPAD = 10240
RPT = NPAD // NS
R_TC = 1024


_sc_mesh = plsc.VectorSubcoreMesh(core_axis_name="c", subcore_axis_name="s")


@functools.partial(
    pl.kernel,
    mesh=_sc_mesh,
    out_type=jax.ShapeDtypeStruct((NC, NPAD), jnp.float32),
    scratch_types=[
        pltpu.VMEM((CPT, B), jnp.int32),
        pltpu.VMEM((B,), jnp.float32),
        pltpu.VMEM_SHARED((NPAD,), jnp.float32),
    ],
)
def _deg_sc(dst_hbm, ones_hbm, zeros1_hbm, out_hbm, dst_v, ones_v, acc_sh):
    c = lax.axis_index("c")
    s = lax.axis_index("s")
    wid = c * NS + s
    row0 = s * RPT
    pltpu.sync_copy(dst_hbm.at[wid], dst_v)
    pltpu.sync_copy(ones_hbm, ones_v)
    pltpu.sync_copy(zeros1_hbm.at[pl.ds(row0, RPT)], acc_sh.at[pl.ds(row0, RPT)])
    plsc.subcore_barrier()

    def chunk(j, carry):
        pltpu.sync_copy(ones_v, acc_sh.at[dst_v.at[j]], add=True)
        return carry

    lax.fori_loop(0, CPT, chunk, 0)
    plsc.subcore_barrier()
    pltpu.sync_copy(acc_sh.at[pl.ds(row0, RPT)], out_hbm.at[c, pl.ds(row0, RPT)])


@functools.partial(
    pl.kernel,
    mesh=_sc_mesh,
    out_type=jax.ShapeDtypeStruct((NC, NPAD, D), jnp.float32),
    scratch_types=[
        pltpu.VMEM((8, B), jnp.int32),
        pltpu.VMEM((8, B), jnp.int32),
        pltpu.VMEM((4, B, D), jnp.float32),
        pltpu.SemaphoreType.DMA,
        pltpu.SemaphoreType.DMA,
        pltpu.SemaphoreType.DMA,
        pltpu.VMEM_SHARED((NPAD, D), jnp.float32),
    ],
)
def _agg_sc(src_hbm, dst_hbm, y_hbm, zeros_hbm, out_hbm,
            src_v, dst_v, rows_v, gsem, isem, ssem, acc_sh):
    c = lax.axis_index("c")
    s = lax.axis_index("s")
    wid = c * NS + s
    row0 = s * RPT

    def load_idx(j):
        buf = lax.rem(j, 8)
        pltpu.make_async_copy(src_hbm.at[wid, j], src_v.at[buf], isem).start()
        pltpu.make_async_copy(dst_hbm.at[wid, j], dst_v.at[buf], isem).start()

    def gather(j):
        pltpu.make_async_copy(y_hbm.at[src_v.at[lax.rem(j, 8)]],
                              rows_v.at[lax.rem(j, 4)], gsem).start()

    def scatter(j):
        pltpu.async_copy(rows_v.at[lax.rem(j, 4)],
                         acc_sh.at[dst_v.at[lax.rem(j, 8)]], ssem, add=True)

    def wait_rows(sem):
        pltpu.make_async_copy(y_hbm.at[pl.ds(0, B)], rows_v.at[0], sem).wait()

    def wait_idx():
        pltpu.make_async_copy(src_hbm.at[0, 0], src_v.at[0], isem).wait()
        pltpu.make_async_copy(dst_hbm.at[0, 0], dst_v.at[0], isem).wait()

    load_idx(0)
    wait_idx()
    gather(0)
    load_idx(1)
    wait_idx()
    gather(1)
    load_idx(2)
    wait_idx()
    gather(2)
    load_idx(3)

    @pl.when(c == 0)
    def _():
        pltpu.sync_copy(y_hbm.at[pl.ds(row0, RPT)], acc_sh.at[pl.ds(row0, RPT)])

    @pl.when(c != 0)
    def _():
        pltpu.sync_copy(zeros_hbm.at[pl.ds(row0, RPT)],
                        acc_sh.at[pl.ds(row0, RPT)])

    plsc.subcore_barrier()

    def chunk(j, carry):
        wait_rows(gsem)
        @pl.when(j > 0)
        def _():
            wait_rows(ssem)
        @pl.when(j + 3 < CPT)
        def _():
            wait_idx()
            gather(j + 3)
        scatter(j)
        @pl.when(j + 4 < CPT)
        def _():
            load_idx(j + 4)
        return carry

    lax.fori_loop(0, CPT, chunk, 0)
    wait_rows(ssem)
    plsc.subcore_barrier()
    pltpu.sync_copy(acc_sh.at[pl.ds(row0, RPT)], out_hbm.at[c, pl.ds(row0, RPT)])


def _prep_body(x_ref, degp_ref, y_ref, dinv_ref):
    deg = degp_ref[0, :] + degp_ref[1, :] + 1.0
    dinv = (1.0 / jnp.sqrt(deg))[:, None]
    y_ref[...] = x_ref[...] * dinv
    dinv_ref[...] = dinv


def _stage_body(p_ref, dinv_ref, w_ref, b_ref, y_ref):
    dinv = dinv_ref[...]
    a = (p_ref[0] + p_ref[1]) * dinv
    h = jnp.dot(a, w_ref[...], preferred_element_type=jnp.float32,
                precision=lax.Precision.HIGHEST) + b_ref[...]
    y_ref[...] = jnp.maximum(h, 0.0) * dinv


def _final_body(p_ref, dinv_ref, w_ref, b_ref, o_ref):
    a = (p_ref[0] + p_ref[1]) * dinv_ref[...]
    o_ref[...] = jnp.dot(a, w_ref[...], preferred_element_type=jnp.float32,
                         precision=lax.Precision.HIGHEST) + b_ref[...]


_GRID = NPAD // R_TC

_prep_tc = pl.pallas_call(
    _prep_body,
    grid=(_GRID,),
    in_specs=[
        pl.BlockSpec((R_TC, D), lambda i: (i, 0)),
        pl.BlockSpec((NC, R_TC), lambda i: (0, i)),
    ],
    out_specs=[
        pl.BlockSpec((R_TC, D), lambda i: (i, 0)),
        pl.BlockSpec((R_TC, 1), lambda i: (i, 0)),
    ],
    out_shape=[
        jax.ShapeDtypeStruct((NPAD, D), jnp.float32),
        jax.ShapeDtypeStruct((NPAD, 1), jnp.float32),
    ],
)

_stage_tc = pl.pallas_call(
    _stage_body,
    grid=(_GRID,),
    in_specs=[
        pl.BlockSpec((NC, R_TC, D), lambda i: (0, i, 0)),
        pl.BlockSpec((R_TC, 1), lambda i: (i, 0)),
        pl.BlockSpec((D, D), lambda i: (0, 0)),
        pl.BlockSpec((1, D), lambda i: (0, 0)),
    ],
    out_specs=pl.BlockSpec((R_TC, D), lambda i: (i, 0)),
    out_shape=jax.ShapeDtypeStruct((NPAD, D), jnp.float32),
)

_final_tc = pl.pallas_call(
    _final_body,
    grid=(_GRID,),
    in_specs=[
        pl.BlockSpec((NC, R_TC, D), lambda i: (0, i, 0)),
        pl.BlockSpec((R_TC, 1), lambda i: (i, 0)),
        pl.BlockSpec((D, D), lambda i: (0, 0)),
        pl.BlockSpec((1, D), lambda i: (0, 0)),
    ],
    out_specs=pl.BlockSpec((R_TC, D), lambda i: (i, 0)),
    out_shape=jax.ShapeDtypeStruct((NPAD, D), jnp.float32),
)


def kernel(x, edge_index, W1, b1, W2, b2, W3, b3, Wmu, bmu, Wls, bls):
    ei = edge_index.astype(jnp.int32)
    n_pad_e = E_PAD - E
    pad_i = jnp.arange(n_pad_e, dtype=jnp.int32)
    pad_src = (pad_i * 37) % N
    pad_dst = N + pad_i % (NPAD - N)
    src3 = jnp.concatenate([ei[0], pad_src]).reshape(NW, CPT, B)
    dst3 = jnp.concatenate([ei[1], pad_dst]).reshape(NW, CPT, B)

    x_pad = jnp.pad(x, ((0, NPAD - N), (0, 0)))
    zeros2 = jnp.zeros((NPAD, D), jnp.float32)
    zeros1 = jnp.zeros((NPAD,), jnp.float32)
    ones_b = jnp.ones((B,), jnp.float32)

    degp = _deg_sc(dst3, ones_b, zeros1)
    y, dinv = _prep_tc(x_pad, degp)

    for W, b in ((W1, b1), (W2, b2), (W3, b3)):
        p = _agg_sc(src3, dst3, y, zeros2)
        y = _stage_tc(p, dinv, W, b.reshape(1, D))

    p = _agg_sc(src3, dst3, y, zeros2)
    Wml = jnp.concatenate([Wmu, Wls], axis=1)
    bml = jnp.concatenate([bmu, bls]).reshape(1, D)
    out = _final_tc(p, dinv, Wml, bml)
    return (out[:N, :D_OUT], out[:N, D_OUT:])


# speedup vs baseline: 1.0226x; 1.0226x over previous
import functools

import jax
import jax.numpy as jnp
from jax import lax
from jax.experimental import pallas as pl
from jax.experimental.pallas import tpu as pltpu
from jax.experimental.pallas import tpu_sc as plsc

N = 10000
E = 320000
D = 128
D_OUT = 64

NC = 2
NS = 16
NW = NC * NS
B = 112
CPT = -(-E // (NW * B))
C_EDGES = CPT * B
E_PAD = NW * C_EDGES
NPAD = 10240
RPT = NPAD // NS
R_TC = 1024


_sc_mesh = plsc.VectorSubcoreMesh(core_axis_name="c", subcore_axis_name="s")


@functools.partial(
    pl.kernel,
    mesh=_sc_mesh,
    out_type=jax.ShapeDtypeStruct((NC, NPAD), jnp.float32),
    scratch_types=[
        pltpu.VMEM((CPT, B), jnp.int32),
        pltpu.VMEM((B,), jnp.float32),
        pltpu.VMEM_SHARED((NPAD,), jnp.float32),
    ],
)
def _deg_sc(dst_hbm, ones_hbm, zeros1_hbm, out_hbm, dst_v, ones_v, acc_sh):
    c = lax.axis_index("c")
    s = lax.axis_index("s")
    wid = c * NS + s
    row0 = s * RPT
    pltpu.sync_copy(dst_hbm.at[wid], dst_v)
    pltpu.sync_copy(ones_hbm, ones_v)
    pltpu.sync_copy(zeros1_hbm.at[pl.ds(row0, RPT)], acc_sh.at[pl.ds(row0, RPT)])
    plsc.subcore_barrier()

    def chunk(j, carry):
        pltpu.sync_copy(ones_v, acc_sh.at[dst_v.at[j]], add=True)
        return carry

    lax.fori_loop(0, CPT, chunk, 0)
    plsc.subcore_barrier()
    pltpu.sync_copy(acc_sh.at[pl.ds(row0, RPT)], out_hbm.at[c, pl.ds(row0, RPT)])


@functools.partial(
    pl.kernel,
    mesh=_sc_mesh,
    out_type=jax.ShapeDtypeStruct((NC, NPAD, D), jnp.float32),
    scratch_types=[
        pltpu.VMEM((8, B), jnp.int32),
        pltpu.VMEM((8, B), jnp.int32),
        pltpu.VMEM((3, B, D), jnp.float32),
        pltpu.SemaphoreType.DMA,
        pltpu.SemaphoreType.DMA,
        pltpu.SemaphoreType.DMA,
        pltpu.VMEM_SHARED((NPAD, D), jnp.float32),
    ],
)
def _agg_sc(src_hbm, dst_hbm, y_hbm, zeros_hbm, out_hbm,
            src_v, dst_v, rows_v, gsem, isem, ssem, acc_sh):
    c = lax.axis_index("c")
    s = lax.axis_index("s")
    wid = c * NS + s
    row0 = s * RPT

    def load_idx(j):
        buf = lax.rem(j, 8)
        pltpu.make_async_copy(src_hbm.at[wid, j], src_v.at[buf], isem).start()
        pltpu.make_async_copy(dst_hbm.at[wid, j], dst_v.at[buf], isem).start()

    def gather(j):
        pltpu.make_async_copy(y_hbm.at[src_v.at[lax.rem(j, 8)]],
                              rows_v.at[lax.rem(j, 3)], gsem).start()

    def scatter(j):
        pltpu.async_copy(rows_v.at[lax.rem(j, 3)],
                         acc_sh.at[dst_v.at[lax.rem(j, 8)]], ssem, add=True)

    def wait_rows(sem):
        pltpu.make_async_copy(y_hbm.at[pl.ds(0, B)], rows_v.at[0], sem).wait()

    def wait_idx():
        pltpu.make_async_copy(src_hbm.at[0, 0], src_v.at[0], isem).wait()
        pltpu.make_async_copy(dst_hbm.at[0, 0], dst_v.at[0], isem).wait()

    load_idx(0)
    wait_idx()
    gather(0)
    load_idx(1)
    wait_idx()
    gather(1)
    load_idx(2)

    @pl.when(c == 0)
    def _():
        pltpu.sync_copy(y_hbm.at[pl.ds(row0, RPT)], acc_sh.at[pl.ds(row0, RPT)])

    @pl.when(c != 0)
    def _():
        pltpu.sync_copy(zeros_hbm.at[pl.ds(row0, RPT)],
                        acc_sh.at[pl.ds(row0, RPT)])

    plsc.subcore_barrier()

    def chunk(j, carry):
        wait_rows(gsem)
        @pl.when(j > 0)
        def _():
            wait_rows(ssem)
        @pl.when(j + 2 < CPT)
        def _():
            wait_idx()
            gather(j + 2)
        scatter(j)
        @pl.when(j + 3 < CPT)
        def _():
            load_idx(j + 3)
        return carry

    lax.fori_loop(0, CPT, chunk, 0)
    wait_rows(ssem)
    plsc.subcore_barrier()
    pltpu.sync_copy(acc_sh.at[pl.ds(row0, RPT)], out_hbm.at[c, pl.ds(row0, RPT)])


def _prep_body(x_ref, degp_ref, y_ref, dinv_ref):
    deg = degp_ref[0, :] + degp_ref[1, :] + 1.0
    dinv = (1.0 / jnp.sqrt(deg))[:, None]
    y_ref[...] = x_ref[...] * dinv
    dinv_ref[...] = dinv


def _stage_body(p_ref, dinv_ref, w_ref, b_ref, y_ref):
    dinv = dinv_ref[...]
    a = (p_ref[0] + p_ref[1]) * dinv
    h = jnp.dot(a, w_ref[...], preferred_element_type=jnp.float32,
                precision=lax.Precision.HIGHEST) + b_ref[...]
    y_ref[...] = jnp.maximum(h, 0.0) * dinv


def _final_body(p_ref, dinv_ref, w_ref, b_ref, o_ref):
    a = (p_ref[0] + p_ref[1]) * dinv_ref[...]
    o_ref[...] = jnp.dot(a, w_ref[...], preferred_element_type=jnp.float32,
                         precision=lax.Precision.HIGHEST) + b_ref[...]


_GRID = NPAD // R_TC

_prep_tc = pl.pallas_call(
    _prep_body,
    grid=(_GRID,),
    in_specs=[
        pl.BlockSpec((R_TC, D), lambda i: (i, 0)),
        pl.BlockSpec((NC, R_TC), lambda i: (0, i)),
    ],
    out_specs=[
        pl.BlockSpec((R_TC, D), lambda i: (i, 0)),
        pl.BlockSpec((R_TC, 1), lambda i: (i, 0)),
    ],
    out_shape=[
        jax.ShapeDtypeStruct((NPAD, D), jnp.float32),
        jax.ShapeDtypeStruct((NPAD, 1), jnp.float32),
    ],
)

_stage_tc = pl.pallas_call(
    _stage_body,
    grid=(_GRID,),
    in_specs=[
        pl.BlockSpec((NC, R_TC, D), lambda i: (0, i, 0)),
        pl.BlockSpec((R_TC, 1), lambda i: (i, 0)),
        pl.BlockSpec((D, D), lambda i: (0, 0)),
        pl.BlockSpec((1, D), lambda i: (0, 0)),
    ],
    out_specs=pl.BlockSpec((R_TC, D), lambda i: (i, 0)),
    out_shape=jax.ShapeDtypeStruct((NPAD, D), jnp.float32),
)

_final_tc = pl.pallas_call(
    _final_body,
    grid=(_GRID,),
    in_specs=[
        pl.BlockSpec((NC, R_TC, D), lambda i: (0, i, 0)),
        pl.BlockSpec((R_TC, 1), lambda i: (i, 0)),
        pl.BlockSpec((D, D), lambda i: (0, 0)),
        pl.BlockSpec((1, D), lambda i: (0, 0)),
    ],
    out_specs=pl.BlockSpec((R_TC, D), lambda i: (i, 0)),
    out_shape=jax.ShapeDtypeStruct((NPAD, D), jnp.float32),
)


def kernel(x, edge_index, W1, b1, W2, b2, W3, b3, Wmu, bmu, Wls, bls):
    ei = edge_index.astype(jnp.int32)
    n_pad_e = E_PAD - E
    pad_i = jnp.arange(n_pad_e, dtype=jnp.int32)
    pad_src = (pad_i * 37) % N
    pad_dst = N + pad_i % (NPAD - N)
    src3 = jnp.concatenate([ei[0], pad_src]).reshape(NW, CPT, B)
    dst3 = jnp.concatenate([ei[1], pad_dst]).reshape(NW, CPT, B)

    x_pad = jnp.pad(x, ((0, NPAD - N), (0, 0)))
    zeros2 = jnp.zeros((NPAD, D), jnp.float32)
    zeros1 = jnp.zeros((NPAD,), jnp.float32)
    ones_b = jnp.ones((B,), jnp.float32)

    degp = _deg_sc(dst3, ones_b, zeros1)
    y, dinv = _prep_tc(x_pad, degp)

    for W, b in ((W1, b1), (W2, b2), (W3, b3)):
        p = _agg_sc(src3, dst3, y, zeros2)
        y = _stage_tc(p, dinv, W, b.reshape(1, D))

    p = _agg_sc(src3, dst3, y, zeros2)
    Wml = jnp.concatenate([Wmu, Wls], axis=1)
    bml = jnp.concatenate([bmu, bls]).reshape(1, D)
    out = _final_tc(p, dinv, Wml, bml)
    return (out[:N, :D_OUT], out[:N, D_OUT:])
